# baseline (speedup 1.0000x reference)
import functools

import jax
import jax.numpy as jnp
from jax import lax
from jax.experimental import pallas as pl
from jax.experimental.pallas import tpu as pltpu
from jax.experimental.pallas import tpu_sc as plsc

_DIM = 64
_NUM_WORKERS = 32
_CHUNK = 128
_NS = 4
_LOOKAHEAD = 3
_LANES = 16


def _build_body(n_t):
  def body(xt_hbm, emb_hbm, out_hbm, idx_v, i2_v, gbuf, tbuf,
           stsem, gsems, ssems):
    c = lax.axis_index("c")
    s = lax.axis_index("s")
    wid = s * 2 + c
    col0 = wid * _CHUNK

    def stage(t, _):
      pltpu.make_async_copy(
          xt_hbm.at[t, pl.ds(col0, _CHUNK)], idx_v.at[t], stsem).start()
      return 0

    lax.fori_loop(0, n_t, stage, 0)

    def drain_stage(t, _):
      pltpu.make_async_copy(
          xt_hbm.at[0, pl.ds(col0, _CHUNK)], idx_v.at[0], stsem).wait()
      return 0

    lax.fori_loop(0, n_t, drain_stage, 0)

    def start_gather(t, b):
      for k in range(8):
        v = idx_v[t, pl.ds(_LANES * k, _LANES)]
        i2_v[b, pl.ds(_LANES * k, _LANES)] = lax.shift_right_logical(v, 1)
      pltpu.make_async_copy(
          emb_hbm.at[i2_v.at[b]],
          gbuf.at[pl.ds(b * _CHUNK, _CHUNK)], gsems.at[b]).start()

    def wait_gather(b):
      pltpu.make_async_copy(
          emb_hbm.at[i2_v.at[0]],
          gbuf.at[pl.ds(0, _CHUNK)], gsems.at[b]).wait()

    def wait_store8(b):
      for _ in range(8):
        pltpu.make_async_copy(
            tbuf.at[pl.ds(0, 8)], out_hbm.at[0, 0, wid], ssems.at[b]).wait()

    bidx = [lax.iota(jnp.int32, _LANES) + _LANES * k for k in range(8)]

    for t in range(_LOOKAHEAD):
      start_gather(t, t)

    def step(t, _):
      b = lax.rem(t, _NS)

      @pl.when(t + _LOOKAHEAD < n_t)
      def _():
        start_gather(t + _LOOKAHEAD, lax.rem(t + _LOOKAHEAD, _NS))

      wait_gather(b)

      @pl.when(t >= _NS)
      def _():
        wait_store8(b)

      rbase = b * _CHUNK
      ridx = [bidx[k] + rbase for k in range(8)]
      par = [
          lax.shift_left(
              lax.bitwise_and(idx_v[t, pl.ds(_LANES * k, _LANES)], 1), 6)
          for k in range(8)
      ]
      trow = b * _DIM

      def band(d_hi, _):
        for d_lo in range(8):
          d = d_hi * 8 + d_lo
          for k in range(8):
            tbuf[trow + d, pl.ds(_LANES * k, _LANES)] = plsc.load_gather(
                gbuf, [ridx[k], par[k] + d])
        pltpu.make_async_copy(
            tbuf.at[pl.ds(trow + d_hi * 8, 8)], out_hbm.at[t, d_hi, wid],
            ssems.at[b]).start()
        return 0

      lax.fori_loop(0, 8, band, 0)
      return 0

    lax.fori_loop(0, n_t, step, 0)

    def drain(b, _):
      wait_store8(b)
      return 0

    lax.fori_loop(0, _NS, drain, 0)

  return body


@functools.partial(jax.jit, static_argnums=())
def _embed(xt, emb2):
  n_t, n_b = xt.shape
  mesh = plsc.VectorSubcoreMesh(core_axis_name="c", subcore_axis_name="s")
  kfn = pl.kernel(
      _build_body(n_t),
      out_type=jax.ShapeDtypeStruct(
          (n_t, 8, _NUM_WORKERS, 8, _CHUNK), jnp.float32),
      mesh=mesh,
      scratch_types=[
          pltpu.VMEM((n_t, _CHUNK), jnp.int32),
          pltpu.VMEM((_NS, _CHUNK), jnp.int32),
          pltpu.VMEM((_NS * _CHUNK, 2 * _DIM), jnp.float32),
          pltpu.VMEM((_NS * _DIM, _CHUNK), jnp.float32),
          pltpu.SemaphoreType.DMA,
          pltpu.SemaphoreType.DMA((_NS,)),
          pltpu.SemaphoreType.DMA((_NS,)),
      ],
      compiler_params=pltpu.CompilerParams(
          use_tc_tiling_on_sc=False, needs_layout_passes=False),
  )
  return kfn(xt, emb2)


def kernel(x, embedding):
  b, t = x.shape
  emb2 = embedding.reshape(-1, 2 * _DIM)
  out5 = _embed(x.T, emb2)
  return out5.transpose(2, 4, 0, 1, 3).reshape(b, t, _DIM)


# speedup vs baseline: 1.6175x; 1.6175x over previous
import functools

import jax
import jax.numpy as jnp
from jax import lax
from jax.experimental import pallas as pl
from jax.experimental.pallas import tpu as pltpu
from jax.experimental.pallas import tpu_sc as plsc

_DIM = 64
_NUM_WORKERS = 32
_CHUNK = 128
_NS = 8
_H = 4


def _build_body(n_t):
  def body(xt_hbm, emb_hbm, out_hbm, idx_v, bufs, stsem, gsems, ssems):
    c = lax.axis_index("c")
    s = lax.axis_index("s")
    wid = s * 2 + c
    col0 = wid * _CHUNK

    def stage(t, _):
      pltpu.make_async_copy(
          xt_hbm.at[t, pl.ds(col0, _CHUNK)], idx_v.at[t], stsem).start()
      return 0

    lax.fori_loop(0, n_t, stage, 0)

    def drain_stage(t, _):
      pltpu.make_async_copy(
          xt_hbm.at[0, pl.ds(col0, _CHUNK)], idx_v.at[0], stsem).wait()
      return 0

    lax.fori_loop(0, n_t, drain_stage, 0)

    def start_gather(t, b):
      pltpu.make_async_copy(
          emb_hbm.at[idx_v.at[t]], bufs[b], gsems[b]).start()

    def wait_gather(b):
      pltpu.make_async_copy(
          emb_hbm.at[idx_v.at[0]], bufs[b], gsems[b]).wait()

    def start_store(t, b):
      pltpu.make_async_copy(bufs[b], out_hbm.at[t, wid], ssems[b]).start()

    def wait_store(b):
      pltpu.make_async_copy(bufs[b], out_hbm.at[0, wid], ssems[b]).wait()

    for t in range(_H):
      start_gather(t, t % _NS)

    for b in range(_NS):
      tg = b + _H
      if tg >= _NS:
        wait_store(tg % _NS)
      start_gather(tg, tg % _NS)
      wait_gather(b)
      start_store(b, b)

    def outer(k, _):
      t0 = k * _NS
      for b in range(_NS):
        t = t0 + b
        bg = (b + _H) % _NS
        wait_store(bg)
        start_gather(t + _H, bg)
        wait_gather(b)
        start_store(t, b)
      return 0

    lax.fori_loop(1, n_t // _NS - 1, outer, 0)

    for b in range(_NS):
      t = n_t - _NS + b
      tg = t + _H
      if tg < n_t:
        bg = tg % _NS
        wait_store(bg)
        start_gather(tg, bg)
      wait_gather(b)
      start_store(t, b)

    for b in range(_NS):
      wait_store(b)

  return body


@functools.partial(jax.jit, static_argnums=())
def _embed(xt, embedding):
  n_t, n_b = xt.shape
  mesh = plsc.VectorSubcoreMesh(core_axis_name="c", subcore_axis_name="s")
  kfn = pl.kernel(
      _build_body(n_t),
      out_type=jax.ShapeDtypeStruct(
          (n_t, _NUM_WORKERS, _CHUNK, _DIM), jnp.float32),
      mesh=mesh,
      scratch_types=[
          pltpu.VMEM((n_t, _CHUNK), jnp.int32),
          [pltpu.VMEM((_CHUNK, _DIM), jnp.float32) for _ in range(_NS)],
          pltpu.SemaphoreType.DMA,
          [pltpu.SemaphoreType.DMA for _ in range(_NS)],
          [pltpu.SemaphoreType.DMA for _ in range(_NS)],
      ],
      compiler_params=pltpu.CompilerParams(use_tc_tiling_on_sc=False),
  )
  return kfn(xt, embedding)


_DT_COLS = 2048


def _detile_body(embt_ref, out_ref):
  y = embt_ref[...].T.reshape(_DT_COLS // 2, 2, _DIM)
  out_ref[:, 0:_DIM] = y[:, 0, :]
  out_ref[:, _DIM:2 * _DIM] = y[:, 1, :]


@jax.jit
def _detile(embt):
  _, v = embt.shape
  grid = (v + _DT_COLS - 1) // _DT_COLS
  return pl.pallas_call(
      _detile_body,
      grid=(grid,),
      in_specs=[pl.BlockSpec((_DIM, _DT_COLS), lambda i: (0, i))],
      out_specs=pl.BlockSpec((_DT_COLS // 2, 2 * _DIM), lambda i: (i, 0)),
      out_shape=jax.ShapeDtypeStruct((v // 2, 2 * _DIM), jnp.float32),
  )(embt)


def kernel(x, embedding):
  b, t = x.shape
  v, d = embedding.shape
  table_lin = _detile(embedding.T).reshape(v, d)
  out = _embed(x.T, table_lin)
  return out.transpose(1, 2, 0, 3).reshape(b, t, _DIM)


# speedup vs baseline: 1.8706x; 1.1565x over previous
import functools

import jax
import jax.numpy as jnp
from jax import lax
from jax.experimental import pallas as pl
from jax.experimental.pallas import tpu as pltpu
from jax.experimental.pallas import tpu_sc as plsc

_DIM = 64
_NUM_WORKERS = 32
_CHUNK = 128
_NS = 8
_H = 4


def _build_body(n_t):
  def body(xt_hbm, emb_hbm, out_hbm, idx_v, bufs, stsem, gsems, ssems):
    c = lax.axis_index("c")
    s = lax.axis_index("s")
    wid = s * 2 + c
    col0 = wid * _CHUNK

    def stage(t, _):
      pltpu.make_async_copy(
          xt_hbm.at[t, pl.ds(col0, _CHUNK)], idx_v.at[t], stsem).start()
      return 0

    lax.fori_loop(0, n_t, stage, 0)

    def drain_stage(t, _):
      pltpu.make_async_copy(
          xt_hbm.at[0, pl.ds(col0, _CHUNK)], idx_v.at[0], stsem).wait()
      return 0

    lax.fori_loop(0, n_t, drain_stage, 0)

    def start_gather(t, b):
      pltpu.make_async_copy(
          emb_hbm.at[idx_v.at[t]], bufs[b], gsems[b]).start()

    def wait_gather(b):
      pltpu.make_async_copy(
          emb_hbm.at[idx_v.at[0]], bufs[b], gsems[b]).wait()

    def start_store(t, b):
      pltpu.make_async_copy(bufs[b], out_hbm.at[t, wid], ssems[b]).start()

    def wait_store(b):
      pltpu.make_async_copy(bufs[b], out_hbm.at[0, wid], ssems[b]).wait()

    for t in range(_H):
      start_gather(t, t % _NS)

    for b in range(_NS):
      tg = b + _H
      if tg >= _NS:
        wait_store(tg % _NS)
      start_gather(tg, tg % _NS)
      wait_gather(b)
      start_store(b, b)

    def outer(k, _):
      t0 = k * _NS
      for b in range(_NS):
        t = t0 + b
        bg = (b + _H) % _NS
        wait_store(bg)
        start_gather(t + _H, bg)
        wait_gather(b)
        start_store(t, b)
      return 0

    lax.fori_loop(1, n_t // _NS - 1, outer, 0)

    for b in range(_NS):
      t = n_t - _NS + b
      tg = t + _H
      if tg < n_t:
        bg = tg % _NS
        wait_store(bg)
        start_gather(tg, bg)
      wait_gather(b)
      start_store(t, b)

    for b in range(_NS):
      wait_store(b)

  return body


@functools.partial(jax.jit, static_argnums=())
def _embed(xt, embedding):
  n_t, n_b = xt.shape
  mesh = plsc.VectorSubcoreMesh(core_axis_name="c", subcore_axis_name="s")
  kfn = pl.kernel(
      _build_body(n_t),
      out_type=jax.ShapeDtypeStruct(
          (n_t, _NUM_WORKERS, _CHUNK, _DIM), jnp.float32),
      mesh=mesh,
      scratch_types=[
          pltpu.VMEM((n_t, _CHUNK), jnp.int32),
          [pltpu.VMEM((_CHUNK, _DIM), jnp.float32) for _ in range(_NS)],
          pltpu.SemaphoreType.DMA,
          [pltpu.SemaphoreType.DMA for _ in range(_NS)],
          [pltpu.SemaphoreType.DMA for _ in range(_NS)],
      ],
      compiler_params=pltpu.CompilerParams(use_tc_tiling_on_sc=False),
  )
  return kfn(xt, embedding)


_DT_COLS = 8192


def _detile_body(embt_ref, out_ref):
  y = embt_ref[...].T.reshape(_DT_COLS // 2, 2, _DIM)
  out_ref[:, 0:_DIM] = y[:, 0, :]
  out_ref[:, _DIM:2 * _DIM] = y[:, 1, :]


@jax.jit
def _detile(embt):
  _, v = embt.shape
  grid = (v + _DT_COLS - 1) // _DT_COLS
  return pl.pallas_call(
      _detile_body,
      grid=(grid,),
      in_specs=[pl.BlockSpec((_DIM, _DT_COLS), lambda i: (0, i))],
      out_specs=pl.BlockSpec((_DT_COLS // 2, 2 * _DIM), lambda i: (i, 0)),
      out_shape=jax.ShapeDtypeStruct((v // 2, 2 * _DIM), jnp.float32),
  )(embt)


def kernel(x, embedding):
  b, t = x.shape
  v, d = embedding.shape
  table_lin = _detile(embedding.T).reshape(v, d)
  out = _embed(x.T, table_lin)
  return out.transpose(1, 2, 0, 3).reshape(b, t, _DIM)


# speedup vs baseline: 1.8842x; 1.0073x over previous
import functools

import jax
import jax.numpy as jnp
from jax import lax
from jax.experimental import pallas as pl
from jax.experimental.pallas import tpu as pltpu
from jax.experimental.pallas import tpu_sc as plsc

_DIM = 64
_NUM_WORKERS = 32
_CHUNK = 128
_NS = 8
_H = 4


def _build_body(n_t):
  def body(xt_hbm, emb_hbm, out_hbm, idx_v, bufs, stsem, gsems, ssems):
    c = lax.axis_index("c")
    s = lax.axis_index("s")
    wid = s * 2 + c
    col0 = wid * _CHUNK

    def stage(t, _):
      pltpu.make_async_copy(
          xt_hbm.at[t, pl.ds(col0, _CHUNK)], idx_v.at[t], stsem).start()
      return 0

    lax.fori_loop(0, n_t, stage, 0)

    def drain_stage(t, _):
      pltpu.make_async_copy(
          xt_hbm.at[0, pl.ds(col0, _CHUNK)], idx_v.at[0], stsem).wait()
      return 0

    lax.fori_loop(0, n_t, drain_stage, 0)

    def start_gather(t, b):
      pltpu.make_async_copy(
          emb_hbm.at[idx_v.at[t]], bufs[b], gsems[b]).start()

    def wait_gather(b):
      pltpu.make_async_copy(
          emb_hbm.at[idx_v.at[0]], bufs[b], gsems[b]).wait()

    def start_store(t, b):
      pltpu.make_async_copy(bufs[b], out_hbm.at[t, wid], ssems[b]).start()

    def wait_store(b):
      pltpu.make_async_copy(bufs[b], out_hbm.at[0, wid], ssems[b]).wait()

    for t in range(_H):
      start_gather(t, t % _NS)

    for b in range(_NS):
      tg = b + _H
      if tg >= _NS:
        wait_store(tg % _NS)
      start_gather(tg, tg % _NS)
      wait_gather(b)
      start_store(b, b)

    def outer(k, _):
      t0 = k * _NS
      for b in range(_NS):
        t = t0 + b
        bg = (b + _H) % _NS
        wait_store(bg)
        start_gather(t + _H, bg)
        wait_gather(b)
        start_store(t, b)
      return 0

    lax.fori_loop(1, n_t // _NS - 1, outer, 0)

    for b in range(_NS):
      t = n_t - _NS + b
      tg = t + _H
      if tg < n_t:
        bg = tg % _NS
        wait_store(bg)
        start_gather(tg, bg)
      wait_gather(b)
      start_store(t, b)

    for b in range(_NS):
      wait_store(b)

  return body


@functools.partial(jax.jit, static_argnums=())
def _embed(xt, embedding):
  n_t, n_b = xt.shape
  mesh = plsc.VectorSubcoreMesh(core_axis_name="c", subcore_axis_name="s")
  kfn = pl.kernel(
      _build_body(n_t),
      out_type=jax.ShapeDtypeStruct(
          (n_t, _NUM_WORKERS, _CHUNK, _DIM), jnp.float32),
      mesh=mesh,
      scratch_types=[
          pltpu.VMEM((n_t, _CHUNK), jnp.int32),
          [pltpu.VMEM((_CHUNK, _DIM), jnp.float32) for _ in range(_NS)],
          pltpu.SemaphoreType.DMA,
          [pltpu.SemaphoreType.DMA for _ in range(_NS)],
          [pltpu.SemaphoreType.DMA for _ in range(_NS)],
      ],
      compiler_params=pltpu.CompilerParams(use_tc_tiling_on_sc=False),
  )
  return kfn(xt, embedding)


_DT_COLS = 16384


def _detile_body(embt_ref, out_ref):
  y = embt_ref[...].T.reshape(_DT_COLS // 2, 2, _DIM)
  out_ref[:, 0:_DIM] = y[:, 0, :]
  out_ref[:, _DIM:2 * _DIM] = y[:, 1, :]


@jax.jit
def _detile(embt):
  _, v = embt.shape
  grid = (v + _DT_COLS - 1) // _DT_COLS
  return pl.pallas_call(
      _detile_body,
      grid=(grid,),
      in_specs=[pl.BlockSpec((_DIM, _DT_COLS), lambda i: (0, i))],
      out_specs=pl.BlockSpec((_DT_COLS // 2, 2 * _DIM), lambda i: (i, 0)),
      out_shape=jax.ShapeDtypeStruct((v // 2, 2 * _DIM), jnp.float32),
  )(embt)


def kernel(x, embedding):
  b, t = x.shape
  v, d = embedding.shape
  table_lin = _detile(embedding.T).reshape(v, d)
  out = _embed(x.T, table_lin)
  return out.transpose(1, 2, 0, 3).reshape(b, t, _DIM)
